# baseline (speedup 1.0000x reference)
import functools

import jax
import jax.numpy as jnp
from jax import lax
from jax.experimental import pallas as pl
from jax.experimental.pallas import tpu as pltpu
import jax.experimental.pallas.tpu_sc as plsc

B = 4096
NUM_WORDS = 200
VOCAB = 100000
EMB = 128
H1 = 64
H2 = 16
EPS = 1e-5

_NC = 2
_NS = 16
_NWORKERS = _NC * _NS

_CH = 128


_H = 2
_HR = _H * _CH


def _sc_gather(table, idx_flat):
    n = idx_flat.shape[0]
    d = table.shape[1]
    per_w = n // _NWORKERS
    n_half = per_w // _HR
    mesh = plsc.VectorSubcoreMesh(core_axis_name="c", subcore_axis_name="s")

    @functools.partial(
        pl.kernel,
        out_type=jax.ShapeDtypeStruct((n, d), table.dtype),
        mesh=mesh,
        scratch_types=[
            pltpu.VMEM((per_w,), jnp.int32),
            pltpu.VMEM((_HR, d), table.dtype),
            pltpu.VMEM((_HR, d), table.dtype),
            pltpu.VMEM((_HR, d), table.dtype),
            pltpu.SemaphoreType.DMA,
            pltpu.SemaphoreType.DMA,
            pltpu.SemaphoreType.DMA,
            pltpu.SemaphoreType.DMA,
            pltpu.SemaphoreType.DMA,
            pltpu.SemaphoreType.DMA,
        ],
    )
    def k(emb_hbm, idx_hbm, out_hbm, idx_v, buf_a, buf_b, buf_c,
          sga, sgb, sgc, swa, swb, swc):
        wid = lax.axis_index("s") * _NC + lax.axis_index("c")
        base = wid * per_w
        bufs = (buf_a, buf_b, buf_c)
        sg = (sga, sgb, sgc)
        sw = (swa, swb, swc)
        pltpu.sync_copy(idx_hbm.at[pl.ds(base, per_w)], idx_v)

        def issue_gather(p, half):
            for j in range(_H):
                pltpu.async_copy(
                    emb_hbm.at[idx_v.at[pl.ds(half * _HR + j * _CH, _CH)]],
                    bufs[p].at[pl.ds(j * _CH, _CH)], sg[p])

        def wait_gather(p):
            for j in range(_H):
                pltpu.make_async_copy(
                    emb_hbm.at[idx_v.at[pl.ds(j * _CH, _CH)]],
                    bufs[p].at[pl.ds(j * _CH, _CH)], sg[p]).wait()

        def issue_wb(p, half):
            pltpu.async_copy(bufs[p], out_hbm.at[pl.ds(base + half * _HR, _HR)], sw[p])

        def wait_wb(p):
            pltpu.make_async_copy(bufs[p], out_hbm.at[pl.ds(base, _HR)], sw[p]).wait()

        def substep(h, p):
            q = (p + 2) % 3

            @pl.when(h >= 1)
            def _():
                wait_wb(q)

            @pl.when(h + 2 < n_half)
            def _():
                issue_gather(q, h + 2)

            wait_gather(p)
            issue_wb(p, h)

        issue_gather(0, 0)
        if n_half > 1:
            issue_gather(1, 1)

        n_main = n_half - (n_half % 3)

        @pl.loop(0, n_main, step=3)
        def _(h):
            substep(h, 0)
            substep(h + 1, 1)
            substep(h + 2, 2)

        for t in range(n_half % 3):
            substep(n_main + t, t)
        wait_wb((n_half - 1) % 3)

    return k(table, idx_flat)


_NCH = 4
_BC = B // _NCH
_BT = 512
_KC = 3200
_NB = _BC // _BT
_NK = (NUM_WORDS * EMB) // _KC
_WC = _KC // EMB


_G = 5


def _mm_body(e_ref, w4_ref, out_ref):
    kk = pl.program_id(1)
    part = jnp.zeros((_BT, H1), dtype=jnp.float32)
    for jw in range(0, _WC, _G):
        ew = jnp.concatenate(
            [e_ref[:, jw + t, :, :].astype(jnp.bfloat16).reshape(_BT, EMB)
             for t in range(_G)], axis=1)
        wg = jnp.concatenate(
            [w4_ref[kk * _WC + jw + t, :, :] for t in range(_G)], axis=0)
        part = part + jnp.dot(ew, wg, preferred_element_type=jnp.float32)

    @pl.when(kk == 0)
    def _():
        out_ref[...] = part

    @pl.when(kk > 0)
    def _():
        out_ref[...] = out_ref[...] + part


def _tc_matmul(e4, w4, interpret=False):
    return pl.pallas_call(
        _mm_body,
        grid=(_NB, _NK),
        in_specs=[
            pl.BlockSpec((_BT // 8, _WC, 8, EMB), lambda b, k: (b, k, 0, 0)),
            pl.BlockSpec((NUM_WORDS, EMB, H1), lambda b, k: (0, 0, 0)),
        ],
        out_specs=pl.BlockSpec((_BT, H1), lambda b, k: (b, 0)),
        out_shape=jax.ShapeDtypeStruct((_BC, H1), jnp.float32),
        interpret=interpret,
    )(e4, w4)


def _tail_body(h1_ref, b1_ref, g1_ref, be1_ref, w2t_ref, b2_ref,
               g2_ref, be2_ref, w3_ref, b3_ref, out_ref):
    h1 = h1_ref[...] + b1_ref[...]
    mu1 = jnp.mean(h1, axis=0, keepdims=True)
    d1 = h1 - mu1
    var1 = jnp.mean(d1 * d1, axis=0, keepdims=True)
    r1 = jnp.maximum(d1 * (g1_ref[...] * lax.rsqrt(var1 + EPS)) + be1_ref[...], 0.0)
    h2 = jnp.dot(r1, w2t_ref[...], preferred_element_type=jnp.float32) + b2_ref[...]
    mu2 = jnp.mean(h2, axis=0, keepdims=True)
    d2 = h2 - mu2
    var2 = jnp.mean(d2 * d2, axis=0, keepdims=True)
    r2 = jnp.maximum(d2 * (g2_ref[...] * lax.rsqrt(var2 + EPS)) + be2_ref[...], 0.0)
    h3 = jnp.sum(r2 * w3_ref[...], axis=1, keepdims=True) + b3_ref[...]
    out_ref[...] = jax.nn.sigmoid(h3)


def _tc_tail(h1, b1, g1, be1, w2t, b2, g2, be2, w3, b3, interpret=False):
    full = lambda s: pl.BlockSpec(s, lambda: tuple(0 for _ in s))
    return pl.pallas_call(
        _tail_body,
        in_specs=[
            full((B, H1)),
            full((1, H1)), full((1, H1)), full((1, H1)),
            full((H1, H2)),
            full((1, H2)), full((1, H2)), full((1, H2)),
            full((1, H2)), full((1, 1)),
        ],
        out_specs=full((B, 1)),
        out_shape=jax.ShapeDtypeStruct((B, 1), jnp.float32),
        interpret=interpret,
    )(h1, b1, g1, be1, w2t, b2, g2, be2, w3, b3)


def kernel(x, emb, W1, b1, g1, be1, W2, b2, g2, be2, W3, b3):
    w4 = W1.reshape(H1, NUM_WORDS, EMB).transpose(1, 2, 0).astype(jnp.bfloat16)
    h1_parts = []
    for c in range(_NCH):
        xc = x[c * _BC:(c + 1) * _BC]
        idx_c = xc.reshape(_BC // 8, 8, NUM_WORDS).transpose(0, 2, 1).reshape(-1)
        e = _sc_gather(emb, idx_c)
        e4 = e.reshape(_BC // 8, NUM_WORDS, 8, EMB)
        h1_parts.append(_tc_matmul(e4, w4))
    h1 = jnp.concatenate(h1_parts, axis=0)
    return _tc_tail(
        h1,
        b1.reshape(1, H1), g1.reshape(1, H1), be1.reshape(1, H1),
        W2.T,
        b2.reshape(1, H2), g2.reshape(1, H2), be2.reshape(1, H2),
        W3.reshape(1, H2), b3.reshape(1, 1),
    )


# speedup vs baseline: 1.0001x; 1.0001x over previous
import functools

import jax
import jax.numpy as jnp
from jax import lax
from jax.experimental import pallas as pl
from jax.experimental.pallas import tpu as pltpu
import jax.experimental.pallas.tpu_sc as plsc

B = 4096
NUM_WORDS = 200
VOCAB = 100000
EMB = 128
H1 = 64
H2 = 16
EPS = 1e-5

_NC = 2
_NS = 16
_NWORKERS = _NC * _NS

_CH = 128


_H = 2
_HR = _H * _CH


def _sc_gather(table, idx_flat):
    n = idx_flat.shape[0]
    d = table.shape[1]
    per_w = n // _NWORKERS
    n_half = per_w // _HR
    mesh = plsc.VectorSubcoreMesh(core_axis_name="c", subcore_axis_name="s")

    @functools.partial(
        pl.kernel,
        out_type=jax.ShapeDtypeStruct((n, d), table.dtype),
        mesh=mesh,
        scratch_types=[
            pltpu.VMEM((per_w,), jnp.int32),
            pltpu.VMEM((_HR, d), table.dtype),
            pltpu.VMEM((_HR, d), table.dtype),
            pltpu.SemaphoreType.DMA,
            pltpu.SemaphoreType.DMA,
            pltpu.SemaphoreType.DMA,
            pltpu.SemaphoreType.DMA,
        ],
    )
    def k(emb_hbm, idx_hbm, out_hbm, idx_v, buf_a, buf_b, sga, sgb, swa, swb):
        wid = lax.axis_index("s") * _NC + lax.axis_index("c")
        base = wid * per_w
        bufs = (buf_a, buf_b)
        sg = (sga, sgb)
        sw = (swa, swb)
        pltpu.sync_copy(idx_hbm.at[pl.ds(base, per_w)], idx_v)

        def issue_gather(p, half):
            for j in range(_H):
                pltpu.async_copy(
                    emb_hbm.at[idx_v.at[pl.ds(half * _HR + j * _CH, _CH)]],
                    bufs[p].at[pl.ds(j * _CH, _CH)], sg[p])

        def wait_gather(p):
            for j in range(_H):
                pltpu.make_async_copy(
                    emb_hbm.at[idx_v.at[pl.ds(j * _CH, _CH)]],
                    bufs[p].at[pl.ds(j * _CH, _CH)], sg[p]).wait()

        def issue_wb(p, half):
            pltpu.async_copy(bufs[p], out_hbm.at[pl.ds(base + half * _HR, _HR)], sw[p])

        def wait_wb(p):
            pltpu.make_async_copy(bufs[p], out_hbm.at[pl.ds(base, _HR)], sw[p]).wait()

        def substep(h, p):
            q = 1 - p

            @pl.when(h >= 1)
            def _():
                wait_wb(q)

            @pl.when(h + 1 < n_half)
            def _():
                issue_gather(q, h + 1)

            wait_gather(p)
            issue_wb(p, h)

        issue_gather(0, 0)

        @pl.loop(0, n_half - (n_half % 2), step=2)
        def _(h):
            substep(h, 0)
            substep(h + 1, 1)

        if n_half % 2:
            substep(n_half - 1, 0)
        wait_wb((n_half - 1) % 2)

    return k(table, idx_flat)


_NCH = 2
_BC = B // _NCH
_BT = 512
_KC = 3200
_NB = _BC // _BT
_NK = (NUM_WORDS * EMB) // _KC
_WC = _KC // EMB


def _mm_body(e_ref, w4_ref, out_ref):
    kk = pl.program_id(1)
    part = jnp.zeros((_BT, H1), dtype=jnp.float32)
    for jw in range(_WC):
        ew = e_ref[:, jw, :, :].astype(jnp.bfloat16).reshape(_BT, EMB)
        part = part + jnp.dot(ew, w4_ref[kk * _WC + jw, :, :],
                              preferred_element_type=jnp.float32)

    @pl.when(kk == 0)
    def _():
        out_ref[...] = part

    @pl.when(kk > 0)
    def _():
        out_ref[...] = out_ref[...] + part


def _tc_matmul(e4, w4, interpret=False):
    return pl.pallas_call(
        _mm_body,
        grid=(_NB, _NK),
        in_specs=[
            pl.BlockSpec((_BT // 8, _WC, 8, EMB), lambda b, k: (b, k, 0, 0)),
            pl.BlockSpec((NUM_WORDS, EMB, H1), lambda b, k: (0, 0, 0)),
        ],
        out_specs=pl.BlockSpec((_BT, H1), lambda b, k: (b, 0)),
        out_shape=jax.ShapeDtypeStruct((_BC, H1), jnp.float32),
        interpret=interpret,
    )(e4, w4)


def _tail_body(h1_ref, b1_ref, g1_ref, be1_ref, w2t_ref, b2_ref,
               g2_ref, be2_ref, w3_ref, b3_ref, out_ref):
    h1 = h1_ref[...] + b1_ref[...]
    mu1 = jnp.mean(h1, axis=0, keepdims=True)
    d1 = h1 - mu1
    var1 = jnp.mean(d1 * d1, axis=0, keepdims=True)
    r1 = jnp.maximum(d1 * (g1_ref[...] * lax.rsqrt(var1 + EPS)) + be1_ref[...], 0.0)
    h2 = jnp.dot(r1, w2t_ref[...], preferred_element_type=jnp.float32) + b2_ref[...]
    mu2 = jnp.mean(h2, axis=0, keepdims=True)
    d2 = h2 - mu2
    var2 = jnp.mean(d2 * d2, axis=0, keepdims=True)
    r2 = jnp.maximum(d2 * (g2_ref[...] * lax.rsqrt(var2 + EPS)) + be2_ref[...], 0.0)
    h3 = jnp.sum(r2 * w3_ref[...], axis=1, keepdims=True) + b3_ref[...]
    out_ref[...] = jax.nn.sigmoid(h3)


def _tc_tail(h1, b1, g1, be1, w2t, b2, g2, be2, w3, b3, interpret=False):
    full = lambda s: pl.BlockSpec(s, lambda: tuple(0 for _ in s))
    return pl.pallas_call(
        _tail_body,
        in_specs=[
            full((B, H1)),
            full((1, H1)), full((1, H1)), full((1, H1)),
            full((H1, H2)),
            full((1, H2)), full((1, H2)), full((1, H2)),
            full((1, H2)), full((1, 1)),
        ],
        out_specs=full((B, 1)),
        out_shape=jax.ShapeDtypeStruct((B, 1), jnp.float32),
        interpret=interpret,
    )(h1, b1, g1, be1, w2t, b2, g2, be2, w3, b3)


def kernel(x, emb, W1, b1, g1, be1, W2, b2, g2, be2, W3, b3):
    w4 = W1.reshape(H1, NUM_WORDS, EMB).transpose(1, 2, 0).astype(jnp.bfloat16)
    h1_parts = []
    for c in range(_NCH):
        xc = x[c * _BC:(c + 1) * _BC]
        idx_c = xc.reshape(_BC // 8, 8, NUM_WORDS).transpose(0, 2, 1).reshape(-1)
        e = _sc_gather(emb, idx_c)
        e4 = e.reshape(_BC // 8, NUM_WORDS, 8, EMB)
        h1_parts.append(_tc_matmul(e4, w4))
    h1 = jnp.concatenate(h1_parts, axis=0)
    return _tc_tail(
        h1,
        b1.reshape(1, H1), g1.reshape(1, H1), be1.reshape(1, H1),
        W2.T,
        b2.reshape(1, H2), g2.reshape(1, H2), be2.reshape(1, H2),
        W3.reshape(1, H2), b3.reshape(1, 1),
    )


# speedup vs baseline: 1.0050x; 1.0049x over previous
import functools

import jax
import jax.numpy as jnp
from jax import lax
from jax.experimental import pallas as pl
from jax.experimental.pallas import tpu as pltpu
import jax.experimental.pallas.tpu_sc as plsc

B = 4096
NUM_WORDS = 200
VOCAB = 100000
EMB = 128
H1 = 64
H2 = 16
EPS = 1e-5

_NC = 2
_NS = 16
_NWORKERS = _NC * _NS

_CH = 128


_H = 2
_HR = _H * _CH


def _sc_gather(table, idx_flat):
    n = idx_flat.shape[0]
    d = table.shape[1]
    per_w = n // _NWORKERS
    n_half = per_w // _HR
    mesh = plsc.VectorSubcoreMesh(core_axis_name="c", subcore_axis_name="s")

    @functools.partial(
        pl.kernel,
        out_type=jax.ShapeDtypeStruct((n, d), table.dtype),
        mesh=mesh,
        scratch_types=[
            pltpu.VMEM((per_w,), jnp.int32),
            pltpu.VMEM((_HR, d), table.dtype),
            pltpu.VMEM((_HR, d), table.dtype),
            pltpu.SemaphoreType.DMA,
            pltpu.SemaphoreType.DMA,
            pltpu.SemaphoreType.DMA,
            pltpu.SemaphoreType.DMA,
        ],
    )
    def k(emb_hbm, idx_hbm, out_hbm, idx_v, buf_a, buf_b, sga, sgb, swa, swb):
        wid = lax.axis_index("s") * _NC + lax.axis_index("c")
        base = wid * per_w
        bufs = (buf_a, buf_b)
        sg = (sga, sgb)
        sw = (swa, swb)
        pltpu.sync_copy(idx_hbm.at[pl.ds(base, per_w)], idx_v)

        def issue_gather(p, half):
            for j in range(_H):
                pltpu.async_copy(
                    emb_hbm.at[idx_v.at[pl.ds(half * _HR + j * _CH, _CH)]],
                    bufs[p].at[pl.ds(j * _CH, _CH)], sg[p])

        def wait_gather(p):
            for j in range(_H):
                pltpu.make_async_copy(
                    emb_hbm.at[idx_v.at[pl.ds(j * _CH, _CH)]],
                    bufs[p].at[pl.ds(j * _CH, _CH)], sg[p]).wait()

        def issue_wb(p, half):
            pltpu.async_copy(bufs[p], out_hbm.at[pl.ds(base + half * _HR, _HR)], sw[p])

        def wait_wb(p):
            pltpu.make_async_copy(bufs[p], out_hbm.at[pl.ds(base, _HR)], sw[p]).wait()

        def substep(h, p):
            q = 1 - p

            @pl.when(h >= 1)
            def _():
                wait_wb(q)

            @pl.when(h + 1 < n_half)
            def _():
                issue_gather(q, h + 1)

            wait_gather(p)
            issue_wb(p, h)

        issue_gather(0, 0)

        @pl.loop(0, n_half - (n_half % 2), step=2)
        def _(h):
            substep(h, 0)
            substep(h + 1, 1)

        if n_half % 2:
            substep(n_half - 1, 0)
        wait_wb((n_half - 1) % 2)

    return k(table, idx_flat)


_NCH = 4
_BC = B // _NCH
_BT = 512
_KC = 3200
_NB = _BC // _BT
_NK = (NUM_WORDS * EMB) // _KC
_WC = _KC // EMB


def _mm_body(e_ref, w4_ref, out_ref):
    kk = pl.program_id(1)
    part = jnp.zeros((_BT, H1), dtype=jnp.float32)
    for jw in range(_WC):
        ew = e_ref[:, jw, :, :].astype(jnp.bfloat16).reshape(_BT, EMB)
        part = part + jnp.dot(ew, w4_ref[kk * _WC + jw, :, :],
                              preferred_element_type=jnp.float32)

    @pl.when(kk == 0)
    def _():
        out_ref[...] = part

    @pl.when(kk > 0)
    def _():
        out_ref[...] = out_ref[...] + part


def _tc_matmul(e4, w4, interpret=False):
    return pl.pallas_call(
        _mm_body,
        grid=(_NB, _NK),
        in_specs=[
            pl.BlockSpec((_BT // 8, _WC, 8, EMB), lambda b, k: (b, k, 0, 0)),
            pl.BlockSpec((NUM_WORDS, EMB, H1), lambda b, k: (0, 0, 0)),
        ],
        out_specs=pl.BlockSpec((_BT, H1), lambda b, k: (b, 0)),
        out_shape=jax.ShapeDtypeStruct((_BC, H1), jnp.float32),
        interpret=interpret,
    )(e4, w4)


def _tail_body(h1_ref, b1_ref, g1_ref, be1_ref, w2t_ref, b2_ref,
               g2_ref, be2_ref, w3_ref, b3_ref, out_ref):
    h1 = h1_ref[...] + b1_ref[...]
    mu1 = jnp.mean(h1, axis=0, keepdims=True)
    d1 = h1 - mu1
    var1 = jnp.mean(d1 * d1, axis=0, keepdims=True)
    r1 = jnp.maximum(d1 * (g1_ref[...] * lax.rsqrt(var1 + EPS)) + be1_ref[...], 0.0)
    h2 = jnp.dot(r1, w2t_ref[...], preferred_element_type=jnp.float32) + b2_ref[...]
    mu2 = jnp.mean(h2, axis=0, keepdims=True)
    d2 = h2 - mu2
    var2 = jnp.mean(d2 * d2, axis=0, keepdims=True)
    r2 = jnp.maximum(d2 * (g2_ref[...] * lax.rsqrt(var2 + EPS)) + be2_ref[...], 0.0)
    h3 = jnp.sum(r2 * w3_ref[...], axis=1, keepdims=True) + b3_ref[...]
    out_ref[...] = jax.nn.sigmoid(h3)


def _tc_tail(h1, b1, g1, be1, w2t, b2, g2, be2, w3, b3, interpret=False):
    full = lambda s: pl.BlockSpec(s, lambda: tuple(0 for _ in s))
    return pl.pallas_call(
        _tail_body,
        in_specs=[
            full((B, H1)),
            full((1, H1)), full((1, H1)), full((1, H1)),
            full((H1, H2)),
            full((1, H2)), full((1, H2)), full((1, H2)),
            full((1, H2)), full((1, 1)),
        ],
        out_specs=full((B, 1)),
        out_shape=jax.ShapeDtypeStruct((B, 1), jnp.float32),
        interpret=interpret,
    )(h1, b1, g1, be1, w2t, b2, g2, be2, w3, b3)


def kernel(x, emb, W1, b1, g1, be1, W2, b2, g2, be2, W3, b3):
    w4 = W1.reshape(H1, NUM_WORDS, EMB).transpose(1, 2, 0).astype(jnp.bfloat16)
    h1_parts = []
    for c in range(_NCH):
        xc = x[c * _BC:(c + 1) * _BC]
        idx_c = xc.reshape(_BC // 8, 8, NUM_WORDS).transpose(0, 2, 1).reshape(-1)
        e = _sc_gather(emb, idx_c)
        e4 = e.reshape(_BC // 8, NUM_WORDS, 8, EMB)
        h1_parts.append(_tc_matmul(e4, w4))
    h1 = jnp.concatenate(h1_parts, axis=0)
    return _tc_tail(
        h1,
        b1.reshape(1, H1), g1.reshape(1, H1), be1.reshape(1, H1),
        W2.T,
        b2.reshape(1, H2), g2.reshape(1, H2), be2.reshape(1, H2),
        W3.reshape(1, H2), b3.reshape(1, 1),
    )
